# Initial kernel scaffold; baseline (speedup 1.0000x reference)
import jax
import jax.numpy as jnp
from jax.experimental import pallas as pl


def kernel(voxel_importance, voxel_coords, voxels, voxel_num_points):
    raise NotImplementedError("write your pallas kernel here")


# speedup vs baseline: 5.7881x; 5.7881x over previous
import functools
import jax
import jax.numpy as jnp
from jax import lax
from jax.experimental import pallas as pl
from jax.experimental.pallas import tpu as pltpu
from jax.experimental.pallas import tpu_sc as plsc

N = 150000
KEEP = 75000
NC = 2
NS = 16
NW = NC * NS
L = 16

NPAD = 150016
CH = NPAD // NS
C16 = CH // L
NREAL_LAST = N - (NS - 1) * CH
RADIX = 256
NBLK = RADIX // L

Q = 2400
VOX_CHUNK = 600


def _sort_body(imp_hbm, outvals_hbm,
               imp_v, keys_v, vals_v, pos_v, hist_v, hist2_v, tsum_v, ts_v,
               base_v, tot_v, bufk0, bufv0, bufk1, bufv1, tsums_sh, sem):
    c = lax.axis_index("c")
    t = lax.axis_index("s")
    lanes = lax.iota(jnp.int32, L)
    zeros = jnp.zeros((L,), jnp.int32)
    ones = jnp.ones((L,), jnp.int32)

    @pl.when(c == 0)
    def _sort():
        ebase = t * CH

        @pl.when(t < NS - 1)
        def _():
            pltpu.sync_copy(imp_hbm.at[pl.ds(ebase, CH)], imp_v)

        @pl.when(t == NS - 1)
        def _():
            pltpu.sync_copy(imp_hbm.at[pl.ds(ebase, NREAL_LAST)],
                            imp_v.at[pl.ds(0, NREAL_LAST)])

        def conv(j, _):
            x = imp_v[pl.ds(j * L, L)]
            u = plsc.bitcast(x, jnp.int32)
            key = jnp.where(u < 0, jnp.bitwise_not(u),
                            u ^ jnp.int32(-2147483648))
            keys_v[pl.ds(j * L, L)] = key
            vals_v[pl.ds(j * L, L)] = ebase + j * L + lanes
            return 0

        lax.fori_loop(0, C16, conv, 0)

        @pl.when(t == NS - 1)
        def _():
            keys_v[pl.ds(NREAL_LAST, L)] = zeros

        pltpu.sync_copy(keys_v, bufk0.at[pl.ds(ebase, CH)])
        pltpu.sync_copy(vals_v, bufv0.at[pl.ds(ebase, CH)])
        plsc.subcore_barrier()

        for p in range(4):
            shift = 8 * p
            bk_in, bv_in = (bufk0, bufv0) if p % 2 == 0 else (bufk1, bufv1)
            bk_out, bv_out = (bufk1, bufv1) if p % 2 == 0 else (bufk0, bufv0)

            pltpu.sync_copy(bk_in.at[pl.ds(ebase, CH)], keys_v)
            pltpu.sync_copy(bv_in.at[pl.ds(ebase, CH)], vals_v)

            def zero(i, _):
                hist_v[pl.ds(i * L, L)] = zeros
                return 0

            lax.fori_loop(0, RADIX, zero, 0)

            def count(j, _):
                idx = lanes * C16 + j
                k = plsc.load_gather(keys_v, [idx])
                d = jnp.bitwise_and(lax.shift_right_logical(k, shift), 255)
                plsc.addupdate_scatter(hist_v, [d * L + lanes], ones)
                return 0

            lax.fori_loop(0, C16, count, 0)

            def transpose(d, _):
                v = hist_v[pl.ds(d * L, L)]
                plsc.store_scatter(hist2_v, [lanes * RADIX + d], v)
                return 0

            lax.fori_loop(0, RADIX, transpose, 0)

            def tsum(blk, _):
                acc = zeros
                for l2 in range(L):
                    acc = acc + hist2_v[pl.ds(l2 * RADIX + blk * L, L)]
                tsum_v[pl.ds(blk * L, L)] = acc
                return 0

            lax.fori_loop(0, NBLK, tsum, 0)
            pltpu.sync_copy(tsum_v, tsums_sh.at[pl.ds(t * RADIX, RADIX)])
            plsc.subcore_barrier()
            pltpu.sync_copy(tsums_sh, ts_v)

            def bases(blk, _):
                accpre = zeros
                tot = zeros
                for t2 in range(NS):
                    v = ts_v[pl.ds(t2 * RADIX + blk * L, L)]
                    tot = tot + v
                    accpre = accpre + jnp.where(t2 < t, v, zeros)
                lacc = accpre
                for l2 in range(L):
                    base_v[pl.ds(l2 * RADIX + blk * L, L)] = lacc
                    lacc = lacc + hist2_v[pl.ds(l2 * RADIX + blk * L, L)]
                tot_v[pl.ds(blk * L, L)] = tot
                return 0

            lax.fori_loop(0, NBLK, bases, 0)

            def gpass(blk, carry):
                tb = tot_v[pl.ds(blk * L, L)]
                g = plsc.cumsum(tb) - tb + carry
                for l2 in range(L):
                    a = pl.ds(l2 * RADIX + blk * L, L)
                    base_v[a] = base_v[a] + g
                return carry + jnp.sum(tb)

            lax.fori_loop(0, NBLK, gpass, jnp.int32(0))

            def rank(j, _):
                idx = lanes * C16 + j
                k = plsc.load_gather(keys_v, [idx])
                d = jnp.bitwise_and(lax.shift_right_logical(k, shift), 255)
                addr = lanes * RADIX + d
                pos = plsc.load_gather(base_v, [addr])
                plsc.store_scatter(base_v, [addr], pos + ones)
                plsc.store_scatter(pos_v, [idx], pos)
                return 0

            lax.fori_loop(0, C16, rank, 0)

            if p < 3:
                cp1 = pltpu.async_copy(keys_v, bk_out.at[pos_v], sem)
                cp2 = pltpu.async_copy(vals_v, bv_out.at[pos_v], sem)
                cp1.wait()
                cp2.wait()
            else:
                pltpu.async_copy(vals_v, outvals_hbm.at[pos_v], sem).wait()
            plsc.subcore_barrier()


@jax.jit
def _sort(imp):
    mesh = plsc.VectorSubcoreMesh(core_axis_name="c", subcore_axis_name="s")
    return pl.kernel(
        _sort_body,
        out_type=jax.ShapeDtypeStruct((NPAD,), jnp.int32),
        mesh=mesh,
        compiler_params=pltpu.CompilerParams(needs_layout_passes=False),
        scratch_types=[
            pltpu.VMEM((CH,), jnp.float32),
            pltpu.VMEM((CH,), jnp.int32),
            pltpu.VMEM((CH,), jnp.int32),
            pltpu.VMEM((CH,), jnp.int32),
            pltpu.VMEM((RADIX * L,), jnp.int32),
            pltpu.VMEM((RADIX * L,), jnp.int32),
            pltpu.VMEM((RADIX,), jnp.int32),
            pltpu.VMEM((NS * RADIX,), jnp.int32),
            pltpu.VMEM((RADIX * L,), jnp.int32),
            pltpu.VMEM((RADIX,), jnp.int32),
            pltpu.VMEM_SHARED((NPAD,), jnp.int32),
            pltpu.VMEM_SHARED((NPAD,), jnp.int32),
            pltpu.VMEM_SHARED((NPAD,), jnp.int32),
            pltpu.VMEM_SHARED((NPAD,), jnp.int32),
            pltpu.VMEM_SHARED((NS * RADIX,), jnp.int32),
            pltpu.SemaphoreType.DMA,
        ],
    )(imp)


def _gather_body(keep_hbm, coords_hbm, vox_hbm, npt_hbm,
                 out_coords, out_vox, out_npt,
                 idx_v, idx4_v, coords_v, npt_v, vox_v, sem):
    c = lax.axis_index("c")
    s = lax.axis_index("s")
    wid = s * NC + c
    base = jnp.minimum(wid * Q, KEEP - Q)
    pltpu.sync_copy(keep_hbm.at[pl.ds(KEEP + 16 + base, Q)], idx_v)
    pltpu.async_copy(npt_hbm.at[idx_v], npt_v, sem).wait()
    pltpu.sync_copy(npt_v, out_npt.at[pl.ds(base, Q)])
    lanes = lax.iota(jnp.int32, L)

    def build(j, _):
        i = j * L + lanes
        q = lax.shift_right_logical(i, 2)
        r = jnp.bitwise_and(i, 3)
        v = plsc.load_gather(idx_v, [q])
        idx4_v[pl.ds(j * L, L)] = v * 4 + r
        return 0

    lax.fori_loop(0, Q * 4 // L, build, 0)
    pltpu.async_copy(coords_hbm.at[idx4_v], coords_v, sem).wait()
    pltpu.sync_copy(coords_v, out_coords.at[pl.ds(base * 4, Q * 4)])
    for ci in range(Q // VOX_CHUNK):
        idx_c = idx_v.at[pl.ds(ci * VOX_CHUNK, VOX_CHUNK)]
        pltpu.async_copy(vox_hbm.at[idx_c], vox_v, sem).wait()
        pltpu.sync_copy(vox_v, out_vox.at[pl.ds(base + ci * VOX_CHUNK, VOX_CHUNK)])


@jax.jit
def _gather(keep, coords_flat, vox, npt):
    mesh = plsc.VectorSubcoreMesh(core_axis_name="c", subcore_axis_name="s")
    return pl.kernel(
        _gather_body,
        out_type=(
            jax.ShapeDtypeStruct((KEEP * 4,), jnp.int32),
            jax.ShapeDtypeStruct((KEEP, 128), jnp.float32),
            jax.ShapeDtypeStruct((KEEP,), jnp.int32),
        ),
        mesh=mesh,
        compiler_params=pltpu.CompilerParams(needs_layout_passes=False),
        scratch_types=[
            pltpu.VMEM((Q,), jnp.int32),
            pltpu.VMEM((Q * 4,), jnp.int32),
            pltpu.VMEM((Q * 4,), jnp.int32),
            pltpu.VMEM((Q,), jnp.int32),
            pltpu.VMEM((VOX_CHUNK, 128), jnp.float32),
            pltpu.SemaphoreType.DMA,
        ],
    )(keep, coords_flat, vox, npt)


def kernel(voxel_importance, voxel_coords, voxels, voxel_num_points):
    order_pad = _sort(voxel_importance.reshape(-1))
    out_c, out_v, out_n = _gather(order_pad, voxel_coords.reshape(-1),
                                  voxels.reshape(N, 128), voxel_num_points)
    return out_c.reshape(KEEP, 4), out_v.reshape(KEEP, 32, 4), out_n


# speedup vs baseline: 6.7869x; 1.1726x over previous
import functools
import jax
import jax.numpy as jnp
from jax import lax
from jax.experimental import pallas as pl
from jax.experimental.pallas import tpu as pltpu
from jax.experimental.pallas import tpu_sc as plsc

N = 150000
KEEP = 75000
NC = 2
NS = 16
NW = NC * NS
L = 16

NPAD = 150016
CH = NPAD // NS
CHH = CH // 2
C16 = CH // L
NREAL_LAST = N - (NS - 1) * CH
RADIX = 2048
NBLK = RADIX // L
HWORDS = RADIX * L
ZCH = HWORDS // NS
SHIFTS = (0, 11, 22)
TSCHUNKS = 4
TSDIG = RADIX // TSCHUNKS

Q = 2400
VOX_CHUNK = 600


def _sort_body(imp_hbm, outvals_hbm,
               imp_v, keys_v, vals_v, pos_v, hb_v, ts_v, tot_v,
               bufk0, bufv0, bufk1, bufv1, tsums_sh, zeros_sh, sem):
    c = lax.axis_index("c")
    t = lax.axis_index("s")
    lanes = lax.iota(jnp.int32, L)
    zeros = jnp.zeros((L,), jnp.int32)
    ones = jnp.ones((L,), jnp.int32)

    @pl.when(c == 0)
    def _sort():
        ebase = t * CH

        def z(i, _):
            keys_v[pl.ds(i * L, L)] = zeros
            return 0

        lax.fori_loop(0, ZCH // L, z, 0, unroll=8)
        pltpu.sync_copy(keys_v.at[pl.ds(0, ZCH)], zeros_sh.at[pl.ds(t * ZCH, ZCH)])

        for h in range(2):
            hoff = h * CHH

            @pl.when(jnp.logical_or(t < NS - 1, h == 0))
            def _():
                pltpu.sync_copy(imp_hbm.at[pl.ds(ebase + hoff, CHH)], imp_v)

            @pl.when(jnp.logical_and(t == NS - 1, h == 1))
            def _():
                pltpu.sync_copy(imp_hbm.at[pl.ds(ebase + hoff, CHH - L)],
                                imp_v.at[pl.ds(0, CHH - L)])

            def conv(j, _):
                x = imp_v[pl.ds(j * L, L)]
                u = plsc.bitcast(x, jnp.int32)
                key = jnp.where(u < 0, jnp.bitwise_not(u),
                                u ^ jnp.int32(-2147483648))
                keys_v[pl.ds(hoff + j * L, L)] = key
                vals_v[pl.ds(hoff + j * L, L)] = ebase + hoff + j * L + lanes
                return 0

            lax.fori_loop(0, CHH // L, conv, 0, unroll=8)

        @pl.when(t == NS - 1)
        def _():
            keys_v[pl.ds(CH - L, L)] = zeros

        pltpu.sync_copy(keys_v, bufk0.at[pl.ds(ebase, CH)])
        pltpu.sync_copy(vals_v, bufv0.at[pl.ds(ebase, CH)])
        plsc.subcore_barrier()

        for p in range(3):
            shift = SHIFTS[p]
            bk_in, bv_in = (bufk0, bufv0) if p % 2 == 0 else (bufk1, bufv1)
            bk_out, bv_out = (bufk1, bufv1) if p % 2 == 0 else (bufk0, bufv0)

            pltpu.sync_copy(bk_in.at[pl.ds(ebase, CH)], keys_v)
            pltpu.sync_copy(bv_in.at[pl.ds(ebase, CH)], vals_v)
            pltpu.sync_copy(zeros_sh, hb_v)

            def count(j, _):
                idx = lanes * C16 + j
                k = plsc.load_gather(keys_v, [idx])
                d = jnp.bitwise_and(lax.shift_right_logical(k, shift), RADIX - 1)
                plsc.addupdate_scatter(hb_v, [lanes * RADIX + d], ones)
                return 0

            lax.fori_loop(0, C16, count, 0, unroll=8)

            def tsum(blk, _):
                acc = zeros
                for l2 in range(L):
                    acc = acc + hb_v[pl.ds(l2 * RADIX + blk * L, L)]
                tsum_slot = pl.ds(blk * L, L)
                tot_v[tsum_slot] = acc
                return 0

            lax.fori_loop(0, NBLK, tsum, 0)
            pltpu.sync_copy(tot_v, tsums_sh.at[pl.ds(t * RADIX, RADIX)])
            plsc.subcore_barrier()

            carry = jnp.int32(0)
            for chunk in range(TSCHUNKS):
                doff = chunk * TSDIG
                for t2 in range(NS):
                    pltpu.sync_copy(
                        tsums_sh.at[pl.ds(t2 * RADIX + doff, TSDIG)],
                        ts_v.at[pl.ds(t2 * TSDIG, TSDIG)])

                def bases(blk, carry):
                    accpre = zeros
                    tb = zeros
                    for t2 in range(NS):
                        v = ts_v[pl.ds(t2 * TSDIG + blk * L, L)]
                        tb = tb + v
                        accpre = accpre + jnp.where(t2 < t, v, zeros)
                    g = plsc.cumsum(tb) - tb + carry
                    lacc = accpre
                    for l2 in range(L):
                        a = pl.ds(l2 * RADIX + doff + blk * L, L)
                        v = hb_v[a]
                        hb_v[a] = lacc + g
                        lacc = lacc + v
                    return carry + jnp.sum(tb)

                carry = lax.fori_loop(0, TSDIG // L, bases, carry)

            def rank(j, _):
                idx = lanes * C16 + j
                k = plsc.load_gather(keys_v, [idx])
                d = jnp.bitwise_and(lax.shift_right_logical(k, shift), RADIX - 1)
                addr = lanes * RADIX + d
                pos = plsc.load_gather(hb_v, [addr])
                plsc.store_scatter(hb_v, [addr], pos + ones)
                plsc.store_scatter(pos_v, [idx], pos)
                return 0

            lax.fori_loop(0, C16, rank, 0, unroll=4)

            if p < 2:
                cp1 = pltpu.async_copy(keys_v, bk_out.at[pos_v], sem)
                cp2 = pltpu.async_copy(vals_v, bv_out.at[pos_v], sem)
                cp1.wait()
                cp2.wait()
            else:
                pltpu.async_copy(vals_v, outvals_hbm.at[pos_v], sem).wait()
            plsc.subcore_barrier()


@jax.jit
def _sort(imp):
    mesh = plsc.VectorSubcoreMesh(core_axis_name="c", subcore_axis_name="s")
    return pl.kernel(
        _sort_body,
        out_type=jax.ShapeDtypeStruct((NPAD,), jnp.int32),
        mesh=mesh,
        compiler_params=pltpu.CompilerParams(needs_layout_passes=False),
        scratch_types=[
            pltpu.VMEM((CHH,), jnp.float32),
            pltpu.VMEM((CH,), jnp.int32),
            pltpu.VMEM((CH,), jnp.int32),
            pltpu.VMEM((CH,), jnp.int32),
            pltpu.VMEM((HWORDS,), jnp.int32),
            pltpu.VMEM((NS * TSDIG,), jnp.int32),
            pltpu.VMEM((RADIX,), jnp.int32),
            pltpu.VMEM_SHARED((NPAD,), jnp.int32),
            pltpu.VMEM_SHARED((NPAD,), jnp.int32),
            pltpu.VMEM_SHARED((NPAD,), jnp.int32),
            pltpu.VMEM_SHARED((NPAD,), jnp.int32),
            pltpu.VMEM_SHARED((NS * RADIX,), jnp.int32),
            pltpu.VMEM_SHARED((HWORDS,), jnp.int32),
            pltpu.SemaphoreType.DMA,
        ],
    )(imp)


def _gather_body(keep_hbm, coords_hbm, vox_hbm, npt_hbm,
                 out_coords, out_vox, out_npt,
                 idx_v, idx4_v, coords_v, npt_v, vox_v, sem):
    c = lax.axis_index("c")
    s = lax.axis_index("s")
    wid = s * NC + c
    base = jnp.minimum(wid * Q, KEEP - Q)
    pltpu.sync_copy(keep_hbm.at[pl.ds(KEEP + 16 + base, Q)], idx_v)
    pltpu.async_copy(npt_hbm.at[idx_v], npt_v, sem).wait()
    pltpu.sync_copy(npt_v, out_npt.at[pl.ds(base, Q)])
    lanes = lax.iota(jnp.int32, L)

    def build(j, _):
        i = j * L + lanes
        q = lax.shift_right_logical(i, 2)
        r = jnp.bitwise_and(i, 3)
        v = plsc.load_gather(idx_v, [q])
        idx4_v[pl.ds(j * L, L)] = v * 4 + r
        return 0

    lax.fori_loop(0, Q * 4 // L, build, 0, unroll=8)
    pltpu.async_copy(coords_hbm.at[idx4_v], coords_v, sem).wait()
    pltpu.sync_copy(coords_v, out_coords.at[pl.ds(base * 4, Q * 4)])
    for ci in range(Q // VOX_CHUNK):
        idx_c = idx_v.at[pl.ds(ci * VOX_CHUNK, VOX_CHUNK)]
        pltpu.async_copy(vox_hbm.at[idx_c], vox_v, sem).wait()
        pltpu.sync_copy(vox_v, out_vox.at[pl.ds(base + ci * VOX_CHUNK, VOX_CHUNK)])


@jax.jit
def _gather(keep, coords_flat, vox, npt):
    mesh = plsc.VectorSubcoreMesh(core_axis_name="c", subcore_axis_name="s")
    return pl.kernel(
        _gather_body,
        out_type=(
            jax.ShapeDtypeStruct((KEEP * 4,), jnp.int32),
            jax.ShapeDtypeStruct((KEEP, 128), jnp.float32),
            jax.ShapeDtypeStruct((KEEP,), jnp.int32),
        ),
        mesh=mesh,
        compiler_params=pltpu.CompilerParams(needs_layout_passes=False),
        scratch_types=[
            pltpu.VMEM((Q,), jnp.int32),
            pltpu.VMEM((Q * 4,), jnp.int32),
            pltpu.VMEM((Q * 4,), jnp.int32),
            pltpu.VMEM((Q,), jnp.int32),
            pltpu.VMEM((VOX_CHUNK, 128), jnp.float32),
            pltpu.SemaphoreType.DMA,
        ],
    )(keep, coords_flat, vox, npt)


def kernel(voxel_importance, voxel_coords, voxels, voxel_num_points):
    order_pad = _sort(voxel_importance.reshape(-1))
    out_c, out_v, out_n = _gather(order_pad, voxel_coords.reshape(-1),
                                  voxels.reshape(N, 128), voxel_num_points)
    return out_c.reshape(KEEP, 4), out_v.reshape(KEEP, 32, 4), out_n


# speedup vs baseline: 6.9530x; 1.0245x over previous
import functools
import jax
import jax.numpy as jnp
from jax import lax
from jax.experimental import pallas as pl
from jax.experimental.pallas import tpu as pltpu
from jax.experimental.pallas import tpu_sc as plsc

N = 150000
KEEP = 75000
NC = 2
NS = 16
NW = NC * NS
L = 16

NPAD = 150016
CH = NPAD // NS
CHH = CH // 2
C16 = CH // L
NREAL_LAST = N - (NS - 1) * CH
IDXBITS = 18
RADIX = 2048
NBLK = RADIX // L
HWORDS = RADIX * L
ZCH = HWORDS // NS
SHIFTS = (0, 11, IDXBITS)
TSCHUNKS = 2
TSDIG = RADIX // TSCHUNKS

Q = 2400
VOX_CHUNK = 600


def _sort_body(imp_hbm, outvals_hbm,
               imp_v, keys_v, vals_v, pos_v, hb_v, ts_v, tot_v,
               bufk0, bufv0, bufk1, bufv1, tsums_sh, zeros_sh, sem):
    c = lax.axis_index("c")
    t = lax.axis_index("s")
    lanes = lax.iota(jnp.int32, L)
    zeros = jnp.zeros((L,), jnp.int32)
    ones = jnp.ones((L,), jnp.int32)

    @pl.when(c == 0)
    def _sort():
        ebase = t * CH

        def z(i, _):
            keys_v[pl.ds(i * L, L)] = zeros
            return 0

        lax.fori_loop(0, ZCH // L, z, 0, unroll=8)
        pltpu.sync_copy(keys_v.at[pl.ds(0, ZCH)], zeros_sh.at[pl.ds(t * ZCH, ZCH)])

        for h in range(2):
            hoff = h * CHH

            @pl.when(jnp.logical_or(t < NS - 1, h == 0))
            def _():
                pltpu.sync_copy(imp_hbm.at[pl.ds(ebase + hoff, CHH)], imp_v)

            @pl.when(jnp.logical_and(t == NS - 1, h == 1))
            def _():
                pltpu.sync_copy(imp_hbm.at[pl.ds(ebase + hoff, CHH - L)],
                                imp_v.at[pl.ds(0, CHH - L)])

            def conv(j, _):
                x = imp_v[pl.ds(j * L, L)]
                u = plsc.bitcast(x, jnp.int32)
                key = jnp.where(u < 0, jnp.bitwise_not(u),
                                u ^ jnp.int32(-2147483648))
                keys_v[pl.ds(hoff + j * L, L)] = key
                vals_v[pl.ds(hoff + j * L, L)] = ebase + hoff + j * L + lanes
                return 0

            lax.fori_loop(0, CHH // L, conv, 0, unroll=8)

        @pl.when(t == NS - 1)
        def _():
            keys_v[pl.ds(CH - L, L)] = zeros

        with jax.named_scope("initpub"):
            pltpu.sync_copy(keys_v, bufk0.at[pl.ds(ebase, CH)])
            pltpu.sync_copy(vals_v, bufv0.at[pl.ds(ebase, CH)])
            plsc.subcore_barrier()

        for p in range(3):
            shift = SHIFTS[p]
            bk_in, bv_in = (bufk0, bufv0) if p % 2 == 0 else (bufk1, bufv1)
            bk_out, bv_out = (bufk1, bufv1) if p % 2 == 0 else (bufk0, bufv0)


            with jax.named_scope("load"):
                pltpu.sync_copy(bk_in.at[pl.ds(ebase, CH)], keys_v)
                if p < 2:
                    pltpu.sync_copy(bv_in.at[pl.ds(ebase, CH)], vals_v)
                pltpu.sync_copy(zeros_sh, hb_v)

            def count(j, _):
                idx = lanes * C16 + j
                k = plsc.load_gather(keys_v, [idx])
                d = jnp.bitwise_and(lax.shift_right_logical(k, shift), RADIX - 1)
                plsc.addupdate_scatter(hb_v, [lanes * RADIX + d], ones)
                return 0

            with jax.named_scope("count"):
                lax.fori_loop(0, C16, count, 0, unroll=8)

            def tsum(blk, _):
                acc = zeros
                for l2 in range(L):
                    acc = acc + hb_v[pl.ds(l2 * RADIX + blk * L, L)]
                tsum_slot = pl.ds(blk * L, L)
                tot_v[tsum_slot] = acc
                return 0

            with jax.named_scope("tsum"):
                lax.fori_loop(0, NBLK, tsum, 0)
                pltpu.sync_copy(tot_v, tsums_sh.at[pl.ds(t * RADIX, RADIX)])
                plsc.subcore_barrier()

            carry = jnp.int32(0)
            for chunk in range(TSCHUNKS):
                doff = chunk * TSDIG
                with jax.named_scope("tstage"):
                    cps = [pltpu.async_copy(
                        tsums_sh.at[pl.ds(t2 * RADIX + doff, TSDIG)],
                        ts_v.at[pl.ds(t2 * TSDIG, TSDIG)], sem)
                        for t2 in range(NS)]
                    for cp in cps:
                        cp.wait()

                def bases(blk, carry):
                    accpre = zeros
                    tb = zeros
                    for t2 in range(NS):
                        v = ts_v[pl.ds(t2 * TSDIG + blk * L, L)]
                        tb = tb + v
                        accpre = accpre + jnp.where(t2 < t, v, zeros)
                    g = plsc.cumsum(tb) - tb + carry
                    lacc = accpre
                    for l2 in range(L):
                        a = pl.ds(l2 * RADIX + doff + blk * L, L)
                        v = hb_v[a]
                        hb_v[a] = lacc + g
                        lacc = lacc + v
                    return carry + jnp.sum(tb)

                with jax.named_scope("bases"):
                    carry = lax.fori_loop(0, TSDIG // L, bases, carry)

            def rank(j, _):
                idx = lanes * C16 + j
                k = plsc.load_gather(keys_v, [idx])
                d = jnp.bitwise_and(lax.shift_right_logical(k, shift), RADIX - 1)
                addr = lanes * RADIX + d
                pos = plsc.load_gather(hb_v, [addr])
                plsc.store_scatter(hb_v, [addr], pos + ones)
                plsc.store_scatter(pos_v, [idx], pos)
                if p == 1:
                    v = plsc.load_gather(vals_v, [idx])
                    pk = jnp.bitwise_or(
                        lax.shift_left(lax.shift_right_logical(k, 22), IDXBITS),
                        v)
                    plsc.store_scatter(keys_v, [idx], pk)
                elif p == 2:
                    plsc.store_scatter(keys_v, [idx],
                                       jnp.bitwise_and(k, (1 << IDXBITS) - 1))
                return 0

            with jax.named_scope("rank"):
                lax.fori_loop(0, C16, rank, 0, unroll=4)

            with jax.named_scope("scatter"):
                if p == 0:
                    cp1 = pltpu.async_copy(keys_v, bk_out.at[pos_v], sem)
                    cp2 = pltpu.async_copy(vals_v, bv_out.at[pos_v], sem)
                    cp1.wait()
                    cp2.wait()
                elif p == 1:
                    pltpu.async_copy(keys_v, bk_out.at[pos_v], sem).wait()
                else:
                    pltpu.async_copy(keys_v, outvals_hbm.at[pos_v], sem).wait()
                plsc.subcore_barrier()


@jax.jit
def _sort(imp):
    mesh = plsc.VectorSubcoreMesh(core_axis_name="c", subcore_axis_name="s")
    return pl.kernel(
        _sort_body,
        out_type=jax.ShapeDtypeStruct((NPAD,), jnp.int32),
        mesh=mesh,
        compiler_params=pltpu.CompilerParams(needs_layout_passes=False),
        scratch_types=[
            pltpu.VMEM((CHH,), jnp.float32),
            pltpu.VMEM((CH,), jnp.int32),
            pltpu.VMEM((CH,), jnp.int32),
            pltpu.VMEM((CH,), jnp.int32),
            pltpu.VMEM((HWORDS,), jnp.int32),
            pltpu.VMEM((NS * TSDIG,), jnp.int32),
            pltpu.VMEM((RADIX,), jnp.int32),
            pltpu.VMEM_SHARED((NPAD,), jnp.int32),
            pltpu.VMEM_SHARED((NPAD,), jnp.int32),
            pltpu.VMEM_SHARED((NPAD,), jnp.int32),
            pltpu.VMEM_SHARED((NPAD,), jnp.int32),
            pltpu.VMEM_SHARED((NS * RADIX,), jnp.int32),
            pltpu.VMEM_SHARED((HWORDS,), jnp.int32),
            pltpu.SemaphoreType.DMA,
        ],
    )(imp)


def _gather_body(keep_hbm, coords_hbm, vox_hbm, npt_hbm,
                 out_coords, out_vox, out_npt,
                 idx_v, idx4_v, coords_v, npt_v, vox_v, sem):
    c = lax.axis_index("c")
    s = lax.axis_index("s")
    wid = s * NC + c
    base = jnp.minimum(wid * Q, KEEP - Q)
    pltpu.sync_copy(keep_hbm.at[pl.ds(KEEP + 16 + base, Q)], idx_v)
    pltpu.async_copy(npt_hbm.at[idx_v], npt_v, sem).wait()
    pltpu.sync_copy(npt_v, out_npt.at[pl.ds(base, Q)])
    lanes = lax.iota(jnp.int32, L)

    def build(j, _):
        i = j * L + lanes
        q = lax.shift_right_logical(i, 2)
        r = jnp.bitwise_and(i, 3)
        v = plsc.load_gather(idx_v, [q])
        idx4_v[pl.ds(j * L, L)] = v * 4 + r
        return 0

    lax.fori_loop(0, Q * 4 // L, build, 0, unroll=8)
    pltpu.async_copy(coords_hbm.at[idx4_v], coords_v, sem).wait()
    pltpu.sync_copy(coords_v, out_coords.at[pl.ds(base * 4, Q * 4)])
    for ci in range(Q // VOX_CHUNK):
        idx_c = idx_v.at[pl.ds(ci * VOX_CHUNK, VOX_CHUNK)]
        pltpu.async_copy(vox_hbm.at[idx_c], vox_v, sem).wait()
        pltpu.sync_copy(vox_v, out_vox.at[pl.ds(base + ci * VOX_CHUNK, VOX_CHUNK)])


@jax.jit
def _gather(keep, coords_flat, vox, npt):
    mesh = plsc.VectorSubcoreMesh(core_axis_name="c", subcore_axis_name="s")
    return pl.kernel(
        _gather_body,
        out_type=(
            jax.ShapeDtypeStruct((KEEP * 4,), jnp.int32),
            jax.ShapeDtypeStruct((KEEP, 128), jnp.float32),
            jax.ShapeDtypeStruct((KEEP,), jnp.int32),
        ),
        mesh=mesh,
        compiler_params=pltpu.CompilerParams(needs_layout_passes=False),
        scratch_types=[
            pltpu.VMEM((Q,), jnp.int32),
            pltpu.VMEM((Q * 4,), jnp.int32),
            pltpu.VMEM((Q * 4,), jnp.int32),
            pltpu.VMEM((Q,), jnp.int32),
            pltpu.VMEM((VOX_CHUNK, 128), jnp.float32),
            pltpu.SemaphoreType.DMA,
        ],
    )(keep, coords_flat, vox, npt)


def kernel(voxel_importance, voxel_coords, voxels, voxel_num_points):
    order_pad = _sort(voxel_importance.reshape(-1))
    out_c, out_v, out_n = _gather(order_pad, voxel_coords.reshape(-1),
                                  voxels.reshape(N, 128), voxel_num_points)
    return out_c.reshape(KEEP, 4), out_v.reshape(KEEP, 32, 4), out_n
